# baseline (speedup 1.0000x reference)
import functools

import jax
import jax.numpy as jnp
from jax import lax
from jax.experimental import pallas as pl
from jax.experimental.pallas import tpu as pltpu
from jax.experimental.pallas import tpu_sc as plsc

B, C, H, W = 16, 32, 128, 512
SLICES = B * C
SLICE_ELEMS = H * W
NUM_CORES, NUM_SUBCORES = 2, 16
NUM_WORKERS = NUM_CORES * NUM_SUBCORES
SLICES_PER_WORKER = SLICES // NUM_WORKERS

CHUNK_ROWS = 16
CHUNK_ELEMS = CHUNK_ROWS * W
CHUNKS = H // CHUNK_ROWS
NBUF = 2
GROUPS = CHUNK_ELEMS // 32
IDX_WORDS = SLICE_ELEMS // 2

XBLK_ROWS = 32
XBLKS = H // XBLK_ROWS
XBLK_NEED = [min((CHUNK_ROWS * c + 29) // XBLK_ROWS, XBLKS - 1)
             for c in range(CHUNKS)]
XBLK_LAST_READER = [min(2 * b + 2, CHUNKS - 1) for b in range(XBLKS)]


@functools.partial(
    pl.kernel,
    out_type=jax.ShapeDtypeStruct((SLICES, H, W), jnp.float32),
    mesh=plsc.VectorSubcoreMesh(core_axis_name="c", subcore_axis_name="s"),
    compiler_params=pltpu.CompilerParams(needs_layout_passes=False),
    scratch_types=[
        pltpu.VMEM((IDX_WORDS,), jnp.int32),
        pltpu.VMEM((H, W), jnp.float32),
        pltpu.VMEM((CHUNK_ROWS, W), jnp.float32),
        pltpu.VMEM((CHUNK_ROWS, W), jnp.float32),
        pltpu.SemaphoreType.DMA,
        pltpu.SemaphoreType.DMA,
        pltpu.SemaphoreType.DMA,
    ],
)
def _sc_gather(
    x_hbm, idx_hbm, out_hbm, idx_v, x_v, out_v0, out_v1, sem0, sem1, sem_x
):
    wid = lax.axis_index("s") * NUM_CORES + lax.axis_index("c")
    s0 = wid * SLICES_PER_WORKER

    def issue_block(s, b):
        pltpu.async_copy(
            x_hbm.at[s, pl.ds(b * XBLK_ROWS, XBLK_ROWS), :],
            x_v.at[pl.ds(b * XBLK_ROWS, XBLK_ROWS), :],
            sem_x,
        )

    def drain_block():
        pltpu.make_async_copy(
            x_hbm.at[0, pl.ds(0, XBLK_ROWS), :],
            x_v.at[pl.ds(0, XBLK_ROWS), :],
            sem_x,
        ).wait()

    pltpu.sync_copy(idx_hbm, idx_v)
    out_bufs = (out_v0, out_v1)
    sems = (sem0, sem1)

    for b in range(XBLKS):
        issue_block(s0, b)

    def do_slice(i, carry):
        s = s0 + i
        sn = jnp.minimum(s + 1, SLICES - 1)

        for c in range(CHUNKS):
            need = XBLK_NEED[c] + 1
            done = (XBLK_NEED[c - 1] + 1) if c else 0
            for _ in range(need - done):
                drain_block()

            buf = out_bufs[c % NBUF]
            drain_out = lambda _buf=buf, _c=c: pltpu.make_async_copy(
                _buf,
                out_hbm.at[s, pl.ds(((_c - NBUF) % CHUNKS) * CHUNK_ROWS, CHUNK_ROWS), :],
                sems[_c % NBUF],
            ).wait()
            if c < NBUF:
                pl.when(i > 0)(drain_out)
            else:
                drain_out()
            idx_base = c * (CHUNK_ELEMS // 2)

            @plsc.parallel_loop(0, GROUPS, unroll=6)
            def do_group(g, _buf=buf, _base=idx_base):
                v = idx_v[pl.ds(_base + g * 16, 16)]
                w0 = v & 0xFFFF
                w1 = lax.shift_right_logical(v, 16)
                r0 = lax.shift_right_logical(w0, 9)
                c0 = w0 & 511
                r1 = lax.shift_right_logical(w1, 9)
                c1 = w1 & 511
                a = plsc.load_gather(x_v, [r0, c0])
                b = plsc.load_gather(x_v, [r1, c1])
                ro = lax.shift_right_logical(g, 4)
                cb = (g & 15) * 32
                _buf[ro, pl.ds(cb, 16)] = a
                _buf[ro, pl.ds(cb + 16, 16)] = b

            pltpu.async_copy(
                buf,
                out_hbm.at[s, pl.ds(c * CHUNK_ROWS, CHUNK_ROWS), :],
                sems[c % NBUF],
            )

            for b in range(XBLKS):
                if XBLK_LAST_READER[b] == c:
                    issue_block(sn, b)

        return carry

    lax.fori_loop(0, SLICES_PER_WORKER, do_slice, 0)

    for c in range(CHUNKS - NBUF, CHUNKS):
        pltpu.make_async_copy(
            out_bufs[c % NBUF],
            out_hbm.at[SLICES - 1, pl.ds(c * CHUNK_ROWS, CHUNK_ROWS), :],
            sems[c % NBUF],
        ).wait()

    for b in range(XBLKS):
        drain_block()


def kernel(x, bn):
    x3 = x.reshape(SLICES, H, W)
    j = jnp.arange(W, dtype=jnp.uint32)
    flat = (bn.astype(jnp.uint32) * jnp.uint32(W) + j[None, :]).reshape(-1, 32)
    packed = flat[:, :16] | (flat[:, 16:] << jnp.uint32(16))
    packed = lax.bitcast_convert_type(packed, jnp.int32).reshape(-1)
    out3 = _sc_gather(x3, packed)
    return out3.reshape(B, C, H, W)


# speedup vs baseline: 1.0273x; 1.0273x over previous
import functools

import jax
import jax.numpy as jnp
from jax import lax
from jax.experimental import pallas as pl
from jax.experimental.pallas import tpu as pltpu
from jax.experimental.pallas import tpu_sc as plsc

B, C, H, W = 16, 32, 128, 512
SLICES = B * C
SLICE_ELEMS = H * W
NUM_CORES, NUM_SUBCORES = 2, 16
NUM_WORKERS = NUM_CORES * NUM_SUBCORES
SLICES_PER_WORKER = SLICES // NUM_WORKERS

CHUNK_ROWS = 16
CHUNK_ELEMS = CHUNK_ROWS * W
CHUNKS = H // CHUNK_ROWS
NBUF = 2
GROUPS = CHUNK_ELEMS // 32
IDX_WORDS = SLICE_ELEMS // 2

XBLK_ROWS = 16
XBLKS = H // XBLK_ROWS
XBLK_NEED = [min((CHUNK_ROWS * c + 29) // XBLK_ROWS, XBLKS - 1)
             for c in range(CHUNKS)]
XBLK_LAST_READER = [min((XBLK_ROWS * (b + 1)) // CHUNK_ROWS, CHUNKS - 1)
                    for b in range(XBLKS)]


@functools.partial(
    pl.kernel,
    out_type=jax.ShapeDtypeStruct((SLICES, H, W), jnp.float32),
    mesh=plsc.VectorSubcoreMesh(core_axis_name="c", subcore_axis_name="s"),
    compiler_params=pltpu.CompilerParams(needs_layout_passes=False),
    scratch_types=[
        pltpu.VMEM((IDX_WORDS,), jnp.int32),
        pltpu.VMEM((H, W), jnp.float32),
        pltpu.VMEM((CHUNK_ROWS, W), jnp.float32),
        pltpu.VMEM((CHUNK_ROWS, W), jnp.float32),
        pltpu.SemaphoreType.DMA,
        pltpu.SemaphoreType.DMA,
        pltpu.SemaphoreType.DMA,
    ],
)
def _sc_gather(
    x_hbm, idx_hbm, out_hbm, idx_v, x_v, out_v0, out_v1, sem0, sem1, sem_x
):
    wid = lax.axis_index("s") * NUM_CORES + lax.axis_index("c")
    s0 = wid * SLICES_PER_WORKER

    def issue_block(s, b):
        pltpu.async_copy(
            x_hbm.at[s, pl.ds(b * XBLK_ROWS, XBLK_ROWS), :],
            x_v.at[pl.ds(b * XBLK_ROWS, XBLK_ROWS), :],
            sem_x,
        )

    def drain_block():
        pltpu.make_async_copy(
            x_hbm.at[0, pl.ds(0, XBLK_ROWS), :],
            x_v.at[pl.ds(0, XBLK_ROWS), :],
            sem_x,
        ).wait()

    pltpu.sync_copy(idx_hbm, idx_v)
    out_bufs = (out_v0, out_v1)
    sems = (sem0, sem1)

    for b in range(XBLKS):
        issue_block(s0, b)

    def do_slice(i, carry):
        s = s0 + i
        sn = jnp.minimum(s + 1, SLICES - 1)

        for c in range(CHUNKS):
            need = XBLK_NEED[c] + 1
            done = (XBLK_NEED[c - 1] + 1) if c else 0
            for _ in range(need - done):
                drain_block()

            buf = out_bufs[c % NBUF]
            drain_out = lambda _buf=buf, _c=c: pltpu.make_async_copy(
                _buf,
                out_hbm.at[s, pl.ds(((_c - NBUF) % CHUNKS) * CHUNK_ROWS, CHUNK_ROWS), :],
                sems[_c % NBUF],
            ).wait()
            if c < NBUF:
                pl.when(i > 0)(drain_out)
            else:
                drain_out()
            idx_base = c * (CHUNK_ELEMS // 2)

            @plsc.parallel_loop(0, GROUPS, unroll=4)
            def do_group(g, _buf=buf, _base=idx_base):
                v = idx_v[pl.ds(_base + g * 16, 16)]
                w0 = v & 0xFFFF
                w1 = lax.shift_right_logical(v, 16)
                r0 = lax.shift_right_logical(w0, 9)
                c0 = w0 & 511
                r1 = lax.shift_right_logical(w1, 9)
                c1 = w1 & 511
                a = plsc.load_gather(x_v, [r0, c0])
                b = plsc.load_gather(x_v, [r1, c1])
                ro = lax.shift_right_logical(g, 4)
                cb = (g & 15) * 32
                _buf[ro, pl.ds(cb, 16)] = a
                _buf[ro, pl.ds(cb + 16, 16)] = b

            pltpu.async_copy(
                buf,
                out_hbm.at[s, pl.ds(c * CHUNK_ROWS, CHUNK_ROWS), :],
                sems[c % NBUF],
            )

            for b in range(XBLKS):
                if XBLK_LAST_READER[b] == c:
                    issue_block(sn, b)

        return carry

    lax.fori_loop(0, SLICES_PER_WORKER, do_slice, 0)

    for c in range(CHUNKS - NBUF, CHUNKS):
        pltpu.make_async_copy(
            out_bufs[c % NBUF],
            out_hbm.at[SLICES - 1, pl.ds(c * CHUNK_ROWS, CHUNK_ROWS), :],
            sems[c % NBUF],
        ).wait()

    for b in range(XBLKS):
        drain_block()


def kernel(x, bn):
    x3 = x.reshape(SLICES, H, W)
    j = jnp.arange(W, dtype=jnp.uint32)
    flat = (bn.astype(jnp.uint32) * jnp.uint32(W) + j[None, :]).reshape(-1, 32)
    packed = flat[:, :16] | (flat[:, 16:] << jnp.uint32(16))
    packed = lax.bitcast_convert_type(packed, jnp.int32).reshape(-1)
    out3 = _sc_gather(x3, packed)
    return out3.reshape(B, C, H, W)


# speedup vs baseline: 1.0302x; 1.0029x over previous
import functools

import jax
import jax.numpy as jnp
from jax import lax
from jax.experimental import pallas as pl
from jax.experimental.pallas import tpu as pltpu
from jax.experimental.pallas import tpu_sc as plsc

B, C, H, W = 16, 32, 128, 512
SLICES = B * C
SLICE_ELEMS = H * W
NUM_CORES, NUM_SUBCORES = 2, 16
NUM_WORKERS = NUM_CORES * NUM_SUBCORES
SLICES_PER_WORKER = SLICES // NUM_WORKERS

CHUNK_ROWS = 16
CHUNK_ELEMS = CHUNK_ROWS * W
CHUNKS = H // CHUNK_ROWS
NBUF = 2
GROUPS = CHUNK_ELEMS // 32
IDX_WORDS = SLICE_ELEMS // 2

XBLK_ROWS = 32
XBLKS = H // XBLK_ROWS
XBLK_NEED = [min((CHUNK_ROWS * c + 29) // XBLK_ROWS, XBLKS - 1)
             for c in range(CHUNKS)]
XBLK_LAST_READER = [min((XBLK_ROWS * (b + 1)) // CHUNK_ROWS, CHUNKS - 1)
                    for b in range(XBLKS)]


@functools.partial(
    pl.kernel,
    out_type=jax.ShapeDtypeStruct((SLICES, H, W), jnp.float32),
    mesh=plsc.VectorSubcoreMesh(core_axis_name="c", subcore_axis_name="s"),
    compiler_params=pltpu.CompilerParams(needs_layout_passes=False),
    scratch_types=[
        pltpu.VMEM((IDX_WORDS,), jnp.int32),
        pltpu.VMEM((H, W), jnp.float32),
        pltpu.VMEM((CHUNK_ROWS, W), jnp.float32),
        pltpu.VMEM((CHUNK_ROWS, W), jnp.float32),
        pltpu.SemaphoreType.DMA,
        pltpu.SemaphoreType.DMA,
        pltpu.SemaphoreType.DMA,
    ],
)
def _sc_gather(
    x_hbm, idx_hbm, out_hbm, idx_v, x_v, out_v0, out_v1, sem0, sem1, sem_x
):
    wid = lax.axis_index("s") * NUM_CORES + lax.axis_index("c")
    s0 = wid * SLICES_PER_WORKER

    def issue_block(s, b):
        pltpu.async_copy(
            x_hbm.at[s, pl.ds(b * XBLK_ROWS, XBLK_ROWS), :],
            x_v.at[pl.ds(b * XBLK_ROWS, XBLK_ROWS), :],
            sem_x,
        )

    def drain_block():
        pltpu.make_async_copy(
            x_hbm.at[0, pl.ds(0, XBLK_ROWS), :],
            x_v.at[pl.ds(0, XBLK_ROWS), :],
            sem_x,
        ).wait()

    pltpu.sync_copy(idx_hbm, idx_v)
    out_bufs = (out_v0, out_v1)
    sems = (sem0, sem1)

    for b in range(XBLKS):
        issue_block(s0, b)

    def do_slice(i, carry):
        s = s0 + i
        sn = jnp.minimum(s + 1, SLICES - 1)

        for c in range(CHUNKS):
            need = XBLK_NEED[c] + 1
            done = (XBLK_NEED[c - 1] + 1) if c else 0
            for _ in range(need - done):
                drain_block()

            buf = out_bufs[c % NBUF]
            drain_out = lambda _buf=buf, _c=c: pltpu.make_async_copy(
                _buf,
                out_hbm.at[s, pl.ds(((_c - NBUF) % CHUNKS) * CHUNK_ROWS, CHUNK_ROWS), :],
                sems[_c % NBUF],
            ).wait()
            if c < NBUF:
                pl.when(i > 0)(drain_out)
            else:
                drain_out()
            idx_base = c * (CHUNK_ELEMS // 2)

            @plsc.parallel_loop(0, GROUPS, unroll=4)
            def do_group(g, _buf=buf, _base=idx_base):
                v = idx_v[pl.ds(_base + g * 16, 16)]
                w0 = v & 0xFFFF
                w1 = lax.shift_right_logical(v, 16)
                r0 = lax.shift_right_logical(w0, 9)
                c0 = w0 & 511
                r1 = lax.shift_right_logical(w1, 9)
                c1 = w1 & 511
                a = plsc.load_gather(x_v, [r0, c0])
                b = plsc.load_gather(x_v, [r1, c1])
                ro = lax.shift_right_logical(g, 4)
                cb = (g & 15) * 32
                _buf[ro, pl.ds(cb, 16)] = a
                _buf[ro, pl.ds(cb + 16, 16)] = b

            pltpu.async_copy(
                buf,
                out_hbm.at[s, pl.ds(c * CHUNK_ROWS, CHUNK_ROWS), :],
                sems[c % NBUF],
            )

            for b in range(XBLKS):
                if XBLK_LAST_READER[b] == c:
                    issue_block(sn, b)

        return carry

    lax.fori_loop(0, SLICES_PER_WORKER, do_slice, 0)

    for c in range(CHUNKS - NBUF, CHUNKS):
        pltpu.make_async_copy(
            out_bufs[c % NBUF],
            out_hbm.at[SLICES - 1, pl.ds(c * CHUNK_ROWS, CHUNK_ROWS), :],
            sems[c % NBUF],
        ).wait()

    for b in range(XBLKS):
        drain_block()


def kernel(x, bn):
    x3 = x.reshape(SLICES, H, W)
    j = jnp.arange(W, dtype=jnp.uint32)
    flat = (bn.astype(jnp.uint32) * jnp.uint32(W) + j[None, :]).reshape(-1, 32)
    packed = flat[:, :16] | (flat[:, 16:] << jnp.uint32(16))
    packed = lax.bitcast_convert_type(packed, jnp.int32).reshape(-1)
    out3 = _sc_gather(x3, packed)
    return out3.reshape(B, C, H, W)
